# initial kernel scaffold (unmeasured)
import jax
import jax.numpy as jnp
from jax import lax
from jax.experimental import pallas as pl
from jax.experimental.pallas import tpu as pltpu


def kernel(
    x,
):
    def body(*refs):
        pass

    out_shape = jax.ShapeDtypeStruct(..., jnp.float32)
    return pl.pallas_call(body, out_shape=out_shape)(...)


# baseline (device time: 102627 ns/iter reference)
import jax
import jax.numpy as jnp
from jax import lax
from jax.experimental import pallas as pl
from jax.experimental.pallas import tpu as pltpu

M, N = 2048, 1024


def kernel(x):
    def body(x_ref, out_ref, rx_ref, ry_ref, send_sems, recv_sems):
        mx = lax.axis_index("x")
        my = lax.axis_index("y")
        x_peer = (1 - mx, my)
        y_peer = (mx, 1 - my)

        barrier_sem = pltpu.get_barrier_semaphore()
        for nbr in (x_peer, y_peer):
            pl.semaphore_signal(
                barrier_sem, inc=1,
                device_id=nbr, device_id_type=pl.DeviceIdType.MESH,
            )
        pl.semaphore_wait(barrier_sem, 2)

        out_ref[:, :] = x_ref[0, 0, :, :].astype(jnp.bfloat16)

        d1 = pltpu.make_async_remote_copy(
            src_ref=out_ref,
            dst_ref=rx_ref,
            send_sem=send_sems.at[0],
            recv_sem=recv_sems.at[0],
            device_id=x_peer,
            device_id_type=pl.DeviceIdType.MESH,
        )
        d1.start()
        d1.wait()
        out_ref[:, :] = out_ref[:, :] + rx_ref[:, :]

        d2 = pltpu.make_async_remote_copy(
            src_ref=out_ref,
            dst_ref=ry_ref,
            send_sem=send_sems.at[1],
            recv_sem=recv_sems.at[1],
            device_id=y_peer,
            device_id_type=pl.DeviceIdType.MESH,
        )
        d2.start()
        d2.wait()
        out_ref[:, :] = out_ref[:, :] + ry_ref[:, :]

    return pl.pallas_call(
        body,
        out_shape=jax.ShapeDtypeStruct((M, N), jnp.bfloat16),
        in_specs=[pl.BlockSpec(memory_space=pltpu.VMEM)],
        out_specs=pl.BlockSpec(memory_space=pltpu.VMEM),
        scratch_shapes=[
            pltpu.VMEM((M, N), jnp.bfloat16),
            pltpu.VMEM((M, N), jnp.bfloat16),
            pltpu.SemaphoreType.DMA((2,)),
            pltpu.SemaphoreType.DMA((2,)),
        ],
        compiler_params=pltpu.CompilerParams(collective_id=0),
    )(x)


# device time: 48208 ns/iter; 2.1288x vs baseline; 2.1288x over previous
import jax
import jax.numpy as jnp
from jax import lax
from jax.experimental import pallas as pl
from jax.experimental.pallas import tpu as pltpu

M, N = 2048, 1024
Q = M // 4


def kernel(x):
    def body(x_ref, out_ref, ra1, rb1, ra2, rb2, ssems, rsems):
        mx = lax.axis_index("x")
        my = lax.axis_index("y")
        x_peer = (1 - mx, my)
        y_peer = (mx, 1 - my)

        a_mine = mx * Q
        a_theirs = (1 - mx) * Q
        b_mine = 2 * Q + my * Q
        b_theirs = 2 * Q + (1 - my) * Q

        barrier_sem = pltpu.get_barrier_semaphore()
        for nbr in (x_peer, y_peer):
            pl.semaphore_signal(
                barrier_sem, inc=1,
                device_id=nbr, device_id_type=pl.DeviceIdType.MESH,
            )
        pl.semaphore_wait(barrier_sem, 2)

        out_ref[:, :] = x_ref[0, 0, :, :].astype(jnp.bfloat16)

        da1 = pltpu.make_async_remote_copy(
            src_ref=out_ref.at[pl.ds(a_theirs, Q)],
            dst_ref=ra1,
            send_sem=ssems.at[0], recv_sem=rsems.at[0],
            device_id=x_peer, device_id_type=pl.DeviceIdType.MESH,
        )
        db1 = pltpu.make_async_remote_copy(
            src_ref=out_ref.at[pl.ds(b_theirs, Q)],
            dst_ref=rb1,
            send_sem=ssems.at[1], recv_sem=rsems.at[1],
            device_id=y_peer, device_id_type=pl.DeviceIdType.MESH,
        )
        da1.start()
        db1.start()
        da1.wait()
        db1.wait()
        out_ref[pl.ds(a_mine, Q), :] = out_ref[pl.ds(a_mine, Q), :] + ra1[:, :]
        out_ref[pl.ds(b_mine, Q), :] = out_ref[pl.ds(b_mine, Q), :] + rb1[:, :]

        da2 = pltpu.make_async_remote_copy(
            src_ref=out_ref.at[pl.ds(a_mine, Q)],
            dst_ref=ra2,
            send_sem=ssems.at[2], recv_sem=rsems.at[2],
            device_id=y_peer, device_id_type=pl.DeviceIdType.MESH,
        )
        db2 = pltpu.make_async_remote_copy(
            src_ref=out_ref.at[pl.ds(b_mine, Q)],
            dst_ref=rb2,
            send_sem=ssems.at[3], recv_sem=rsems.at[3],
            device_id=x_peer, device_id_type=pl.DeviceIdType.MESH,
        )
        da2.start()
        db2.start()
        da2.wait()
        db2.wait()
        out_ref[pl.ds(a_mine, Q), :] = out_ref[pl.ds(a_mine, Q), :] + ra2[:, :]
        out_ref[pl.ds(b_mine, Q), :] = out_ref[pl.ds(b_mine, Q), :] + rb2[:, :]

        sa3 = pltpu.make_async_remote_copy(
            src_ref=out_ref.at[pl.ds(a_mine, Q)],
            dst_ref=out_ref.at[pl.ds(a_mine, Q)],
            send_sem=ssems.at[4], recv_sem=rsems.at[4],
            device_id=x_peer, device_id_type=pl.DeviceIdType.MESH,
        )
        sb3 = pltpu.make_async_remote_copy(
            src_ref=out_ref.at[pl.ds(b_mine, Q)],
            dst_ref=out_ref.at[pl.ds(b_mine, Q)],
            send_sem=ssems.at[5], recv_sem=rsems.at[5],
            device_id=y_peer, device_id_type=pl.DeviceIdType.MESH,
        )
        sa3.start()
        sb3.start()
        ra3 = pltpu.make_async_remote_copy(
            src_ref=out_ref.at[pl.ds(a_mine, Q)],
            dst_ref=out_ref.at[pl.ds(a_theirs, Q)],
            send_sem=ssems.at[4], recv_sem=rsems.at[4],
            device_id=x_peer, device_id_type=pl.DeviceIdType.MESH,
        )
        rb3 = pltpu.make_async_remote_copy(
            src_ref=out_ref.at[pl.ds(b_mine, Q)],
            dst_ref=out_ref.at[pl.ds(b_theirs, Q)],
            send_sem=ssems.at[5], recv_sem=rsems.at[5],
            device_id=y_peer, device_id_type=pl.DeviceIdType.MESH,
        )
        ra3.wait_recv()
        rb3.wait_recv()
        sa3.wait_send()
        sb3.wait_send()

    return pl.pallas_call(
        body,
        out_shape=jax.ShapeDtypeStruct((M, N), jnp.bfloat16),
        in_specs=[pl.BlockSpec(memory_space=pltpu.VMEM)],
        out_specs=pl.BlockSpec(memory_space=pltpu.VMEM),
        scratch_shapes=[
            pltpu.VMEM((Q, N), jnp.bfloat16),
            pltpu.VMEM((Q, N), jnp.bfloat16),
            pltpu.VMEM((Q, N), jnp.bfloat16),
            pltpu.VMEM((Q, N), jnp.bfloat16),
            pltpu.SemaphoreType.DMA((6,)),
            pltpu.SemaphoreType.DMA((6,)),
        ],
        compiler_params=pltpu.CompilerParams(collective_id=0),
    )(x)


# device time: 44326 ns/iter; 2.3153x vs baseline; 1.0876x over previous
import jax
import jax.numpy as jnp
from jax import lax
from jax.experimental import pallas as pl
from jax.experimental.pallas import tpu as pltpu

M, N = 2048, 1024
Q = M // 4
NC = 2
CW = N // NC

BF16 = jnp.bfloat16


def kernel(x):
    def sem(phase, stream, c):
        return phase * (2 * NC) + stream * NC + c

    def body(x_ref, out_ref, ra1, rb1, ra2, rb2, ssems, rsems):
        mx = lax.axis_index("x")
        my = lax.axis_index("y")
        x_peer = (1 - mx, my)
        y_peer = (mx, 1 - my)

        a_mine = mx * Q
        a_theirs = (1 - mx) * Q
        b_mine = 2 * Q + my * Q
        b_theirs = 2 * Q + (1 - my) * Q

        def copy(src, dst, phase, stream, c, peer):
            return pltpu.make_async_remote_copy(
                src_ref=src, dst_ref=dst,
                send_sem=ssems.at[sem(phase, stream, c)],
                recv_sem=rsems.at[sem(phase, stream, c)],
                device_id=peer, device_id_type=pl.DeviceIdType.MESH,
            )

        barrier_sem = pltpu.get_barrier_semaphore()
        for nbr in (x_peer, y_peer):
            pl.semaphore_signal(
                barrier_sem, inc=1,
                device_id=nbr, device_id_type=pl.DeviceIdType.MESH,
            )
        pl.semaphore_wait(barrier_sem, 2)

        out_ref[pl.ds(a_theirs, Q), :] = x_ref[
            0, 0, pl.ds(a_theirs, Q), :].astype(BF16)
        out_ref[pl.ds(b_theirs, Q), :] = x_ref[
            0, 0, pl.ds(b_theirs, Q), :].astype(BF16)

        p1a = [
            copy(out_ref.at[pl.ds(a_theirs, Q), pl.ds(c * CW, CW)],
                 ra1.at[:, pl.ds(c * CW, CW)], 0, 0, c, x_peer)
            for c in range(NC)
        ]
        p1b = [
            copy(out_ref.at[pl.ds(b_theirs, Q), pl.ds(c * CW, CW)],
                 rb1.at[:, pl.ds(c * CW, CW)], 0, 1, c, y_peer)
            for c in range(NC)
        ]
        for d in p1a + p1b:
            d.start()

        out_ref[pl.ds(a_mine, Q), :] = x_ref[
            0, 0, pl.ds(a_mine, Q), :].astype(BF16)
        out_ref[pl.ds(b_mine, Q), :] = x_ref[
            0, 0, pl.ds(b_mine, Q), :].astype(BF16)

        p2a, p2b = [], []
        for c in range(NC):
            cols = pl.ds(c * CW, CW)
            p1a[c].wait_recv()
            out_ref[pl.ds(a_mine, Q), cols] = (
                out_ref[pl.ds(a_mine, Q), cols] + ra1[:, cols]
            )
            d = copy(out_ref.at[pl.ds(a_mine, Q), cols],
                     ra2.at[:, cols], 1, 0, c, y_peer)
            d.start()
            p2a.append(d)

            p1b[c].wait_recv()
            out_ref[pl.ds(b_mine, Q), cols] = (
                out_ref[pl.ds(b_mine, Q), cols] + rb1[:, cols]
            )
            d = copy(out_ref.at[pl.ds(b_mine, Q), cols],
                     rb2.at[:, cols], 1, 1, c, x_peer)
            d.start()
            p2b.append(d)

        p3a, p3b = [], []
        for c in range(NC):
            cols = pl.ds(c * CW, CW)
            p2a[c].wait()
            out_ref[pl.ds(a_mine, Q), cols] = (
                out_ref[pl.ds(a_mine, Q), cols] + ra2[:, cols]
            )
            d = copy(out_ref.at[pl.ds(a_mine, Q), cols],
                     out_ref.at[pl.ds(a_mine, Q), cols], 2, 0, c, x_peer)
            d.start()
            p3a.append(d)

            p2b[c].wait()
            out_ref[pl.ds(b_mine, Q), cols] = (
                out_ref[pl.ds(b_mine, Q), cols] + rb2[:, cols]
            )
            d = copy(out_ref.at[pl.ds(b_mine, Q), cols],
                     out_ref.at[pl.ds(b_mine, Q), cols], 2, 1, c, y_peer)
            d.start()
            p3b.append(d)

        for c in range(NC):
            cols = pl.ds(c * CW, CW)
            copy(out_ref.at[pl.ds(a_mine, Q), cols],
                 out_ref.at[pl.ds(a_theirs, Q), cols],
                 2, 0, c, x_peer).wait_recv()
            copy(out_ref.at[pl.ds(b_mine, Q), cols],
                 out_ref.at[pl.ds(b_theirs, Q), cols],
                 2, 1, c, y_peer).wait_recv()

        for d in p1a + p1b + p3a + p3b:
            d.wait_send()

    return pl.pallas_call(
        body,
        out_shape=jax.ShapeDtypeStruct((M, N), BF16),
        in_specs=[pl.BlockSpec(memory_space=pltpu.VMEM)],
        out_specs=pl.BlockSpec(memory_space=pltpu.VMEM),
        scratch_shapes=[
            pltpu.VMEM((Q, N), BF16),
            pltpu.VMEM((Q, N), BF16),
            pltpu.VMEM((Q, N), BF16),
            pltpu.VMEM((Q, N), BF16),
            pltpu.SemaphoreType.DMA((3 * 2 * NC,)),
            pltpu.SemaphoreType.DMA((3 * 2 * NC,)),
        ],
        compiler_params=pltpu.CompilerParams(collective_id=0),
    )(x)
